# baseline (speedup 1.0000x reference)
import jax
import jax.numpy as jnp
from jax.experimental import pallas as pl


def _copy_body(x_ref, o_ref):
    o_ref[0, 0, 0] = x_ref[0, 0]


def kernel(x):
    b, s, n, m, d = x.shape
    J, K = 8, 7
    grid = (b, K, J)

    def in_map(bi, k, j):
        row = jax.lax.select(k < 6, k, 6 + j)
        return (bi, row, 0, 0, 0)

    def out_map(bi, k, j):
        return (bi, j, k, 0, 0, 0)

    return pl.pallas_call(
        _copy_body,
        grid=grid,
        in_specs=[pl.BlockSpec((1, 1, n, m, d), in_map)],
        out_specs=pl.BlockSpec((1, 1, 1, n, m, d), out_map),
        out_shape=jax.ShapeDtypeStruct((b, J, K, n, m, d), x.dtype),
    )(x)


# speedup vs baseline: 1.5311x; 1.5311x over previous
import jax
import jax.numpy as jnp
from jax.experimental import pallas as pl


def _copy_body(x_ref, o_ref):
    j = pl.program_id(1)
    o_ref[0, 0, :6] = x_ref[0, :6]
    o_ref[0, 0, 6] = x_ref[0, 6 + j]


def kernel(x):
    b, s, n, m, d = x.shape
    J, K = 8, 7
    grid = (b, J)

    return pl.pallas_call(
        _copy_body,
        grid=grid,
        in_specs=[pl.BlockSpec((1, s, n, m, d), lambda bi, j: (bi, 0, 0, 0, 0))],
        out_specs=pl.BlockSpec((1, 1, K, n, m, d), lambda bi, j: (bi, j, 0, 0, 0, 0)),
        out_shape=jax.ShapeDtypeStruct((b, J, K, n, m, d), x.dtype),
    )(x)


# speedup vs baseline: 1.7022x; 1.1117x over previous
import jax
import jax.numpy as jnp
from jax.experimental import pallas as pl
from jax.experimental.pallas import tpu as pltpu


def _copy_body(x_ref, o_ref, sem):
    bi = pl.program_id(0)
    copies = []
    for j in range(8):
        c1 = pltpu.make_async_copy(x_ref.at[0, :6], o_ref.at[bi, j, :6], sem)
        c2 = pltpu.make_async_copy(x_ref.at[0, 6 + j], o_ref.at[bi, j, 6], sem)
        c1.start()
        c2.start()
        copies.append(c1)
        copies.append(c2)
    for c in copies:
        c.wait()


def kernel(x):
    b, s, n, m, d = x.shape
    J, K = 8, 7

    return pl.pallas_call(
        _copy_body,
        grid=(b,),
        in_specs=[pl.BlockSpec((1, s, n, m, d), lambda bi: (bi, 0, 0, 0, 0))],
        out_specs=pl.BlockSpec(memory_space=pl.ANY),
        out_shape=jax.ShapeDtypeStruct((b, J, K, n, m, d), x.dtype),
        scratch_shapes=[pltpu.SemaphoreType.DMA],
    )(x)
